# Initial kernel scaffold; baseline (speedup 1.0000x reference)
import jax
import jax.numpy as jnp
from jax.experimental import pallas as pl


def kernel(feature, edge_index, edge_type, W1, b1, W2, b2):
    raise NotImplementedError("write your pallas kernel here")


# speedup vs baseline: 31.5690x; 31.5690x over previous
import functools

import jax
import jax.numpy as jnp
import numpy as np
from jax import lax
from jax.experimental import pallas as pl
from jax.experimental.pallas import tpu as pltpu
from jax.experimental.pallas import tpu_sc as plsc

N = 100000
E = 3200000
NPAD = 100352
SUB = 128
EROWS = E // SUB
NWORK = 32
RPT = (EROWS // NWORK) // 8 * 8
NMACRO = RPT // 8
EXT_BASE = NWORK * RPT
EXT_TILES = (EROWS - EXT_BASE) // 8
NPT = NPAD // 16

_ZEROS16 = np.zeros((NPAD, 16), np.float32)
_ZEROS4 = np.zeros((NPAD, 4), np.float32)
_ONES4 = np.ones((SUB, 4), np.float32)


def _make_deg_kernel():
    mesh = plsc.VectorSubcoreMesh(core_axis_name="c", subcore_axis_name="s")

    @functools.partial(
        pl.kernel, mesh=mesh,
        compiler_params=pltpu.CompilerParams(use_tc_tiling_on_sc=False),
        out_type=jax.ShapeDtypeStruct((2, NPAD, 4), jnp.float32),
        scratch_types=[
            pltpu.VMEM((8, SUB), jnp.int32),
            pltpu.VMEM((SUB, 4), jnp.float32),
            pltpu.VMEM_SHARED((NPAD, 4), jnp.float32),
        ],
    )
    def deg_kernel(dst_hbm, ones_hbm, zeros_hbm, out_hbm, dst_v, ones_v, acc):
        cid = lax.axis_index("c")
        sid = lax.axis_index("s")
        wid = sid * 2 + cid
        pltpu.sync_copy(zeros_hbm.at[pl.ds(sid * NPT, NPT), :],
                        acc.at[pl.ds(sid * NPT, NPT), :])
        pltpu.sync_copy(ones_hbm, ones_v)
        plsc.subcore_barrier()

        def run_macro(r0):
            pltpu.sync_copy(dst_hbm.at[pl.ds(r0, 8), :], dst_v)
            for j in range(8):
                pltpu.sync_copy(ones_v, acc.at[dst_v.at[j]], add=True)

        def macro(m, carry):
            run_macro(wid * RPT + m * 8)
            return carry

        lax.fori_loop(0, NMACRO, macro, 0)

        @pl.when(wid < EXT_TILES)
        def _extra():
            run_macro(EXT_BASE + wid * 8)

        plsc.subcore_barrier()
        pltpu.sync_copy(acc.at[pl.ds(sid * NPT, NPT), :],
                        out_hbm.at[cid, pl.ds(sid * NPT, NPT), :])

    return deg_kernel


def _make_prop_kernel(width):
    mesh = plsc.VectorSubcoreMesh(core_axis_name="c", subcore_axis_name="s")

    @functools.partial(
        pl.kernel, mesh=mesh,
        compiler_params=pltpu.CompilerParams(use_tc_tiling_on_sc=False),
        out_type=jax.ShapeDtypeStruct((2, NPAD, width), jnp.float32),
        scratch_types=[
            pltpu.VMEM((8, SUB), jnp.int32),
            pltpu.VMEM((8, SUB), jnp.int32),
            pltpu.VMEM((8 * SUB, width), jnp.float32),
            pltpu.VMEM_SHARED((NPAD, width), jnp.float32),
            pltpu.SemaphoreType.DMA,
        ],
    )
    def prop_kernel(src_hbm, dst_hbm, z_hbm, zeros_hbm, out_hbm,
                    src_v, dst_v, rows_v, acc, sem):
        cid = lax.axis_index("c")
        sid = lax.axis_index("s")
        wid = sid * 2 + cid
        pltpu.sync_copy(zeros_hbm.at[pl.ds(sid * NPT, NPT), :],
                        acc.at[pl.ds(sid * NPT, NPT), :])
        plsc.subcore_barrier()

        def run_macro(r0):
            pltpu.sync_copy(src_hbm.at[pl.ds(r0, 8), :], src_v)
            pltpu.sync_copy(dst_hbm.at[pl.ds(r0, 8), :], dst_v)
            for j in range(8):
                pltpu.async_copy(z_hbm.at[src_v.at[j]],
                                 rows_v.at[pl.ds(j * SUB, SUB), :], sem).wait()
                pltpu.sync_copy(rows_v.at[pl.ds(j * SUB, SUB), :],
                                acc.at[dst_v.at[j]], add=True)

        def macro(m, carry):
            run_macro(wid * RPT + m * 8)
            return carry

        lax.fori_loop(0, NMACRO, macro, 0)

        @pl.when(wid < EXT_TILES)
        def _extra():
            run_macro(EXT_BASE + wid * 8)

        plsc.subcore_barrier()
        pltpu.sync_copy(acc.at[pl.ds(sid * NPT, NPT), :],
                        out_hbm.at[cid, pl.ds(sid * NPT, NPT), :])

    return prop_kernel


_deg = _make_deg_kernel()
_prop16 = _make_prop_kernel(16)
_prop4 = _make_prop_kernel(4)

BR = 2048
GRID = NPAD // BR


def _prep_body(dega_ref, degb_ref, x_ref, dis_ref, z1_ref):
    deg = dega_ref[...][:, 0:1] + degb_ref[...][:, 0:1] + 1.0
    dis = lax.rsqrt(deg)
    dis_ref[...] = dis
    z1_ref[...] = x_ref[...] * dis


def _mid_body(s1a_ref, s1b_ref, z1_ref, dis_ref, w1_ref, b1_ref, w2_ref, z2_ref):
    dis = dis_ref[...]
    t = (s1a_ref[...] + s1b_ref[...] + z1_ref[...]) * dis
    h = jnp.maximum(
        jnp.dot(t, w1_ref[...], preferred_element_type=jnp.float32) + b1_ref[...],
        0.0)
    g = jnp.dot(h, w2_ref[...], preferred_element_type=jnp.float32)
    z2_ref[...] = g * dis


def _fin_body(s2a_ref, s2b_ref, z2_ref, dis_ref, b2_ref, out_ref):
    s = s2a_ref[...] + s2b_ref[...] + z2_ref[...]
    out_ref[...] = s * dis_ref[...] + b2_ref[...]


def _row_spec(w):
    return pl.BlockSpec((BR, w), lambda i: (i, 0))


def _full_spec(shape):
    return pl.BlockSpec(shape, lambda i: tuple(0 for _ in shape))


def kernel(feature, edge_index, edge_type, W1, b1, W2, b2):
    f32 = jnp.float32
    src2d = edge_index[0].astype(jnp.int32).reshape(EROWS, SUB)
    dst2d = edge_index[1].astype(jnp.int32).reshape(EROWS, SUB)
    x = jnp.pad(feature.astype(f32), ((0, NPAD - N), (0, 0)))
    zeros16 = jnp.asarray(_ZEROS16)
    zeros4 = jnp.asarray(_ZEROS4)
    ones4 = jnp.asarray(_ONES4)
    w2p = jnp.pad(W2.astype(f32), ((0, 0), (0, 1)))
    b1r = b1.astype(f32).reshape(1, 64)
    b2p = jnp.pad(b2.astype(f32), (0, 1)).reshape(1, 4)

    degp = _deg(dst2d, ones4, zeros4)

    dis, z1 = pl.pallas_call(
        _prep_body,
        grid=(GRID,),
        in_specs=[_row_spec(4), _row_spec(4), _row_spec(16)],
        out_specs=[_row_spec(1), _row_spec(16)],
        out_shape=[jax.ShapeDtypeStruct((NPAD, 1), f32),
                   jax.ShapeDtypeStruct((NPAD, 16), f32)],
    )(degp[0], degp[1], x)

    s1p = _prop16(src2d, dst2d, z1, zeros16)

    z2 = pl.pallas_call(
        _mid_body,
        grid=(GRID,),
        in_specs=[_row_spec(16), _row_spec(16), _row_spec(16), _row_spec(1),
                  _full_spec((16, 64)), _full_spec((1, 64)), _full_spec((64, 4))],
        out_specs=_row_spec(4),
        out_shape=jax.ShapeDtypeStruct((NPAD, 4), f32),
    )(s1p[0], s1p[1], z1, dis, W1.astype(f32), b1r, w2p)

    s2p = _prop4(src2d, dst2d, z2, zeros4)

    outp = pl.pallas_call(
        _fin_body,
        grid=(GRID,),
        in_specs=[_row_spec(4), _row_spec(4), _row_spec(4), _row_spec(1),
                  _full_spec((1, 4))],
        out_specs=_row_spec(4),
        out_shape=jax.ShapeDtypeStruct((NPAD, 4), f32),
    )(s2p[0], s2p[1], z2, dis, b2p)

    return outp[:N, :3]


# speedup vs baseline: 89.2583x; 2.8274x over previous
import functools

import jax
import jax.numpy as jnp
import numpy as np
from jax import lax
from jax.experimental import pallas as pl
from jax.experimental.pallas import tpu as pltpu
from jax.experimental.pallas import tpu_sc as plsc

N = 100000
E = 3200000
NPAD = 100352
K8 = NPAD // 8
SUB = 128
EROWS = E // SUB
NWORK = 32
MROWS = 4
RPT = (EROWS // NWORK) // 8 * 8
NMACRO = RPT // MROWS
LOOPN = (NMACRO - 2) // 4
EXT_BASE = NWORK * RPT
EXT_TILES = (EROWS - EXT_BASE) // 8
NPT = NPAD // 16
RNODE = NPAD // 32
PCH = 448
WCH = 224
assert NMACRO % 4 == 2 and LOOPN * 4 + 2 == NMACRO
assert NPT % WCH == 0 and RNODE % PCH == 0

_ZEROS16 = np.zeros((NPAD, 16), np.float32)
_ZEROS4 = np.zeros((NPAD, 4), np.float32)
_ONES4 = np.ones((SUB, 4), np.float32)


def _make_deg_kernel():
    mesh = plsc.VectorSubcoreMesh(core_axis_name="c", subcore_axis_name="s")

    @functools.partial(
        pl.kernel, mesh=mesh,
        compiler_params=pltpu.CompilerParams(use_tc_tiling_on_sc=False, needs_layout_passes=False),
        out_type=jax.ShapeDtypeStruct((2, NPAD, 4), jnp.float32),
        scratch_types=[
            pltpu.VMEM((4, MROWS, SUB), jnp.int32),
            pltpu.VMEM((SUB, 4), jnp.float32),
            pltpu.VMEM_SHARED((NPAD, 4), jnp.float32),
            pltpu.SemaphoreType.DMA,
            pltpu.SemaphoreType.DMA,
            pltpu.SemaphoreType.DMA,
            pltpu.SemaphoreType.DMA,
        ],
    )
    def deg_kernel(e_hbm, ones_hbm, zeros_hbm, out_hbm, dst_v, ones_v, acc,
                   six0, six1, ss0, ss1):
        cid = lax.axis_index("c")
        sid = lax.axis_index("s")
        wid = sid * 2 + cid
        six = (six0, six1)
        ss = (ss0, ss1)
        pltpu.sync_copy(zeros_hbm.at[pl.ds(sid * NPT, NPT), :],
                        acc.at[pl.ds(sid * NPT, NPT), :])
        pltpu.sync_copy(ones_hbm, ones_v)
        plsc.subcore_barrier()

        def base(m):
            return wid * RPT + m * MROWS

        def fire_idx(r0, slot, p):
            pltpu.async_copy(e_hbm.at[1, pl.ds(r0, MROWS), :], dst_v.at[slot],
                             six[p])

        def drain_idx(slot, p):
            pltpu.make_async_copy(e_hbm.at[1, pl.ds(0, MROWS), :],
                                  dst_v.at[slot], six[p]).wait()

        def fire_scat(slot, p):
            for j in range(MROWS):
                pltpu.async_copy(ones_v, acc.at[dst_v.at[slot, j]], ss[p],
                                 add=True)

        def drain_scat(slot, p):
            for j in range(MROWS):
                pltpu.make_async_copy(ones_v, acc.at[dst_v.at[slot, j]],
                                      ss[p]).wait()

        fire_idx(base(0), 0, 0)
        fire_idx(base(1), 1, 1)

        def body(i, carry):
            for k in range(4):
                p = k & 1
                drain_idx(k, p)
                fire_scat(k, p)
                if k == 0:
                    @pl.when(i > 0)
                    def _():
                        drain_scat(3, 1)
                else:
                    drain_scat(k - 1, 1 - p)
                fire_idx(base(4 * i + k + 2), (k + 2) % 4, p)
            return carry

        lax.fori_loop(0, LOOPN, body, 0)
        drain_idx(0, 0)
        fire_scat(0, 0)
        drain_scat(3, 1)
        drain_idx(1, 1)
        fire_scat(1, 1)
        drain_scat(0, 0)
        drain_scat(1, 1)

        @pl.when(wid < EXT_TILES)
        def _extra():
            e0 = EXT_BASE + wid * 8
            fire_idx(e0, 2, 0)
            drain_idx(2, 0)
            fire_scat(2, 0)
            fire_idx(e0 + MROWS, 3, 1)
            drain_idx(3, 1)
            fire_scat(3, 1)
            drain_scat(2, 0)
            drain_scat(3, 1)

        plsc.subcore_barrier()
        pltpu.sync_copy(acc.at[pl.ds(sid * NPT, NPT), :],
                        out_hbm.at[cid, pl.ds(sid * NPT, NPT), :])

    return deg_kernel


def _make_prep_kernel():
    mesh = plsc.VectorSubcoreMesh(core_axis_name="c", subcore_axis_name="s")

    @functools.partial(
        pl.kernel, mesh=mesh,
        compiler_params=pltpu.CompilerParams(use_tc_tiling_on_sc=False, needs_layout_passes=False),
        out_type=[jax.ShapeDtypeStruct((NPAD, 16), jnp.float32),
                  jax.ShapeDtypeStruct((K8, 128), jnp.float32)],
        scratch_types=[
            pltpu.VMEM((PCH, 4), jnp.float32),
            pltpu.VMEM((PCH, 4), jnp.float32),
            pltpu.VMEM((PCH, 16), jnp.float32),
            pltpu.VMEM((PCH, 16), jnp.float32),
            pltpu.VMEM((PCH // 8, 128), jnp.float32),
            pltpu.VMEM((16,), jnp.float32),
        ],
    )
    def prep_kernel(degp_hbm, x_hbm, z1_hbm, disw_hbm,
                    dp0_v, dp1_v, x_v, z1_v, dw_v, sbuf):
        cid = lax.axis_index("c")
        sid = lax.axis_index("s")
        wid = sid * 2 + cid
        nbase = wid * RNODE

        iota = lax.iota(jnp.int32, 16)
        ridx0 = lax.shift_right_logical(iota, 2)
        cidx = lax.bitwise_and(iota, 3)
        magic = jnp.full((16,), 0x5F3759DF, jnp.int32)
        c15 = jnp.full((16,), 1.5, jnp.float32)

        def rsqrt16(x):
            h = x * 0.5
            i = plsc.bitcast(x, jnp.int32)
            i = magic - lax.shift_right_logical(i, 1)
            y = plsc.bitcast(i, jnp.float32)
            y = y * (c15 - h * y * y)
            y = y * (c15 - h * y * y)
            y = y * (c15 - h * y * y)
            return y

        def process_chunk(r0, count):
            pltpu.sync_copy(degp_hbm.at[0, pl.ds(r0, count), :],
                            dp0_v.at[pl.ds(0, count), :])
            pltpu.sync_copy(degp_hbm.at[1, pl.ds(r0, count), :],
                            dp1_v.at[pl.ds(0, count), :])
            pltpu.sync_copy(x_hbm.at[pl.ds(r0, count), :],
                            x_v.at[pl.ds(0, count), :])

            def group(g, carry):
                rbase = g * 4
                ridx = ridx0 + rbase
                d0 = plsc.load_gather(dp0_v, [ridx, cidx])
                d1 = plsc.load_gather(dp1_v, [ridx, cidx])
                dis4 = rsqrt16(d0 + d1 + 1.0)
                sbuf[...] = dis4
                for j in range(4):
                    lane = jnp.full((16,), 4 * j, jnp.int32)
                    splat = plsc.load_gather(sbuf, [lane])
                    row = x_v[rbase + j, :]
                    z1_v[rbase + j, :] = row * splat
                    rr = lax.div(rbase + j, 8)
                    cc = lax.rem(rbase + j, 8)
                    plsc.store_scatter(dw_v, [jnp.full((16,), rr, jnp.int32),
                                              cc * 16 + iota], splat)
                return carry

            lax.fori_loop(0, count // 4, group, 0)
            pltpu.sync_copy(z1_v.at[pl.ds(0, count), :],
                            z1_hbm.at[pl.ds(r0, count), :])
            pltpu.sync_copy(dw_v.at[pl.ds(0, count // 8), :],
                            disw_hbm.at[pl.ds(r0 // 8, count // 8), :])

        for c in range(7):
            if c < 6:
                process_chunk(nbase + c * PCH, PCH)
            else:
                @pl.when(wid < 31)
                def _():
                    process_chunk(nbase + 6 * PCH, PCH)

        @pl.when(wid == 31)
        def _tail():
            r0 = 31 * RNODE + 6 * PCH
            process_chunk(r0, N - (31 * RNODE + 6 * PCH))

    return prep_kernel


def _make_prop_kernel():
    mesh = plsc.VectorSubcoreMesh(core_axis_name="c", subcore_axis_name="s")

    @functools.partial(
        pl.kernel, mesh=mesh,
        compiler_params=pltpu.CompilerParams(use_tc_tiling_on_sc=False, needs_layout_passes=False),
        out_type=jax.ShapeDtypeStruct((2, K8, 128), jnp.float32),
        scratch_types=[
            pltpu.VMEM((4, MROWS, SUB), jnp.int32),
            pltpu.VMEM((4, MROWS, SUB), jnp.int32),
            pltpu.VMEM((2, MROWS * SUB, 16), jnp.float32),
            pltpu.VMEM((WCH, 16), jnp.float32),
            pltpu.VMEM((WCH // 8, 128), jnp.float32),
            pltpu.VMEM_SHARED((NPAD, 16), jnp.float32),
            pltpu.SemaphoreType.DMA,
            pltpu.SemaphoreType.DMA,
            pltpu.SemaphoreType.DMA,
            pltpu.SemaphoreType.DMA,
            pltpu.SemaphoreType.DMA,
        ],
    )
    def prop_kernel(e_hbm, z_hbm, zeros_hbm, out_hbm,
                    src_v, dst_v, rows_v, a_v, b_v, acc,
                    six0, six1, sg, ss0, ss1):
        cid = lax.axis_index("c")
        sid = lax.axis_index("s")
        wid = sid * 2 + cid
        six = (six0, six1)
        ss = (ss0, ss1)

        @pl.when(cid == 0)
        def _():
            pltpu.sync_copy(z_hbm.at[pl.ds(sid * NPT, NPT), :],
                            acc.at[pl.ds(sid * NPT, NPT), :])

        @pl.when(cid == 1)
        def _():
            pltpu.sync_copy(zeros_hbm.at[pl.ds(sid * NPT, NPT), :],
                            acc.at[pl.ds(sid * NPT, NPT), :])

        plsc.subcore_barrier()

        def base(m):
            return wid * RPT + m * MROWS

        def fire_idx(r0, slot, p):
            pltpu.async_copy(e_hbm.at[0, pl.ds(r0, MROWS), :], src_v.at[slot],
                             six[p])
            pltpu.async_copy(e_hbm.at[1, pl.ds(r0, MROWS), :], dst_v.at[slot],
                             six[p])

        def drain_idx(slot, p):
            pltpu.make_async_copy(e_hbm.at[0, pl.ds(0, MROWS), :],
                                  src_v.at[slot], six[p]).wait()
            pltpu.make_async_copy(e_hbm.at[1, pl.ds(0, MROWS), :],
                                  dst_v.at[slot], six[p]).wait()

        def gathers(slot, p):
            ds_list = [
                pltpu.async_copy(z_hbm.at[src_v.at[slot, j]],
                                 rows_v.at[p, pl.ds(j * SUB, SUB), :], sg)
                for j in range(MROWS)
            ]
            for d in ds_list:
                d.wait()

        def fire_scat(slot, p):
            for j in range(MROWS):
                pltpu.async_copy(rows_v.at[p, pl.ds(j * SUB, SUB), :],
                                 acc.at[dst_v.at[slot, j]], ss[p], add=True)

        def drain_scat(slot, p):
            for j in range(MROWS):
                pltpu.make_async_copy(rows_v.at[p, pl.ds(j * SUB, SUB), :],
                                      acc.at[dst_v.at[slot, j]], ss[p]).wait()

        fire_idx(base(0), 0, 0)
        fire_idx(base(1), 1, 1)

        def body(i, carry):
            for k in range(4):
                p = k & 1
                drain_idx(k, p)
                gathers(k, p)
                fire_scat(k, p)
                if k == 0:
                    @pl.when(i > 0)
                    def _():
                        drain_scat(3, 1)
                else:
                    drain_scat(k - 1, 1 - p)
                fire_idx(base(4 * i + k + 2), (k + 2) % 4, p)
            return carry

        lax.fori_loop(0, LOOPN, body, 0)
        drain_idx(0, 0)
        gathers(0, 0)
        fire_scat(0, 0)
        drain_scat(3, 1)
        drain_idx(1, 1)
        gathers(1, 1)
        fire_scat(1, 1)
        drain_scat(0, 0)
        drain_scat(1, 1)

        @pl.when(wid < EXT_TILES)
        def _extra():
            e0 = EXT_BASE + wid * 8
            fire_idx(e0, 2, 0)
            drain_idx(2, 0)
            gathers(2, 0)
            fire_scat(2, 0)
            fire_idx(e0 + MROWS, 3, 1)
            drain_idx(3, 1)
            gathers(3, 1)
            fire_scat(3, 1)
            drain_scat(2, 0)
            drain_scat(3, 1)

        plsc.subcore_barrier()

        def wchunk(c, carry):
            r0 = sid * NPT + c * WCH
            pltpu.sync_copy(acc.at[pl.ds(r0, WCH), :], a_v)

            def rrow(rr, carry2):
                for ccj in range(8):
                    b_v[rr, pl.ds(ccj * 16, 16)] = a_v[rr * 8 + ccj, :]
                return carry2

            lax.fori_loop(0, WCH // 8, rrow, 0)
            pltpu.sync_copy(b_v, out_hbm.at[cid, pl.ds(r0 // 8, WCH // 8), :])
            return carry

        lax.fori_loop(0, NPT // WCH, wchunk, 0)

    return prop_kernel


_deg = _make_deg_kernel()
_prep = _make_prep_kernel()
_prop = _make_prop_kernel()

BRW = 256
GRIDW = K8 // BRW


def _mid_body(s1w_ref, disw_ref, w1_ref, b1_ref, w2_ref, z2w_ref):
    s = s1w_ref[...]
    dis = disw_ref[...]
    t = (s[0] + s[1]) * dis
    h = jnp.maximum(
        jnp.dot(t, w1_ref[...], preferred_element_type=jnp.float32)
        + b1_ref[...], 0.0)
    z2w_ref[...] = jnp.dot(h, w2_ref[...],
                           preferred_element_type=jnp.float32) * dis


def _fin_body(s2w_ref, disw_ref, b2_ref, outw_ref):
    s = s2w_ref[...]
    outw_ref[...] = (s[0] + s[1]) * disw_ref[...] + b2_ref[...]


def _row_spec(w):
    return pl.BlockSpec((BRW, w), lambda i: (i, 0))


def _full_spec(shape):
    return pl.BlockSpec(shape, lambda i: tuple(0 for _ in shape))


def kernel(feature, edge_index, edge_type, W1, b1, W2, b2):
    f32 = jnp.float32
    e3d = edge_index.astype(jnp.int32).reshape(2, EROWS, SUB)
    x = feature.astype(f32)
    zeros16 = jnp.asarray(_ZEROS16)
    zeros4 = jnp.asarray(_ZEROS4)
    ones4 = jnp.asarray(_ONES4)
    eye8 = jnp.eye(8, dtype=f32)
    w1bd = jnp.kron(eye8, W1.astype(f32))
    b1w = jnp.tile(b1.astype(f32), 8).reshape(1, 512)
    w2p = jnp.pad(W2.astype(f32), ((0, 0), (0, 13)))
    w2bd = jnp.kron(eye8, w2p)
    b2w = jnp.tile(jnp.pad(b2.astype(f32), (0, 13)), 8).reshape(1, 128)

    degp = _deg(e3d, ones4, zeros4)
    z1, disw = _prep(degp, x)
    s1w = _prop(e3d, z1, zeros16)

    z2w = pl.pallas_call(
        _mid_body,
        grid=(GRIDW,),
        in_specs=[pl.BlockSpec((2, BRW, 128), lambda i: (0, i, 0)),
                  _row_spec(128),
                  _full_spec((128, 512)), _full_spec((1, 512)),
                  _full_spec((512, 128))],
        out_specs=_row_spec(128),
        out_shape=jax.ShapeDtypeStruct((K8, 128), f32),
    )(s1w, disw, w1bd, b1w, w2bd)

    z2n = jnp.reshape(z2w, (NPAD, 16))
    s2w = _prop(e3d, z2n, zeros16)

    outw = pl.pallas_call(
        _fin_body,
        grid=(GRIDW,),
        in_specs=[pl.BlockSpec((2, BRW, 128), lambda i: (0, i, 0)),
                  _row_spec(128), _full_spec((1, 128))],
        out_specs=_row_spec(128),
        out_shape=jax.ShapeDtypeStruct((K8, 128), f32),
    )(s2w, disw, b2w)

    return jnp.reshape(outw, (NPAD, 16))[:N, :3]
